# Initial kernel scaffold; baseline (speedup 1.0000x reference)
import jax
import jax.numpy as jnp
from jax.experimental import pallas as pl


def kernel(feats, W1, b1, W2, b2, W3, b3):
    raise NotImplementedError("write your pallas kernel here")


# speedup vs baseline: 64.6530x; 64.6530x over previous
import functools

import jax
import jax.numpy as jnp
from jax.experimental import pallas as pl
from jax.experimental.pallas import tpu as pltpu

_RESOLUTIONS = (17, 33, 65, 129)
_NUM_POINTS = (0, 8000, 8000, 8000)
_FINAL = 129
_BLK = 512


def _mlp_body(pts_ref, feats_ref, w1p_ref, w1f_ref, b1_ref, w2_ref, b2_ref,
              w3_ref, b3_ref, out_ref):
    c1 = jnp.dot(feats_ref[...], w1f_ref[...],
                 preferred_element_type=jnp.float32) + b1_ref[...]
    h = jnp.dot(pts_ref[...], w1p_ref[...],
                preferred_element_type=jnp.float32) + c1
    h = jnp.maximum(h, 0.0)
    h = jnp.dot(h, w2_ref[...], preferred_element_type=jnp.float32) + b2_ref[...]
    h = jnp.maximum(h, 0.0)
    o = jnp.dot(h, w3_ref[...], preferred_element_type=jnp.float32) + b3_ref[...]
    out_ref[...] = jax.nn.sigmoid(o)


def _mlp_eval(pts_pad, feats, w1p, w1f, b1, w2, b2, w3p, b3p):
    n_pad = pts_pad.shape[0]
    grid = (n_pad // _BLK,)
    full = lambda shape: pl.BlockSpec(shape, lambda i: (0, 0))
    return pl.pallas_call(
        _mlp_body,
        grid=grid,
        in_specs=[
            pl.BlockSpec((_BLK, 128), lambda i: (i, 0)),
            full((1, 256)), full((128, 256)), full((256, 256)),
            full((1, 256)), full((256, 256)), full((1, 256)),
            full((256, 128)), full((1, 128)),
        ],
        out_specs=pl.BlockSpec((_BLK, 128), lambda i: (i, 0)),
        out_shape=jax.ShapeDtypeStruct((n_pad, 128), jnp.float32),
    )(pts_pad, feats, w1p, w1f, b1, w2, b2, w3p, b3p)


def _pad_points(coords3d):
    n = coords3d.shape[0]
    n_pad = (n + _BLK - 1) // _BLK * _BLK
    pts = jnp.zeros((n_pad, 128), jnp.float32)
    return pts.at[:n, :3].set(coords3d)


def _scale(coords):
    c = coords.astype(jnp.float32) / (_FINAL - 1)
    return c * 2.0 - 1.0


def _upsample(vol):
    def up_last(v):
        d = v.shape[-1]
        mid = 0.5 * v[..., :-1] + 0.5 * v[..., 1:]
        stacked = jnp.stack([v[..., :-1], mid], axis=-1).reshape(v.shape[:-1] + (2 * (d - 1),))
        return jnp.concatenate([stacked, v[..., -1:]], axis=-1)
    v = up_last(vol)
    v = up_last(v.transpose(0, 2, 1)).transpose(0, 2, 1)
    v = up_last(v.transpose(2, 1, 0)).transpose(2, 1, 0)
    return v


def kernel(feats, W1, b1, W2, b2, W3, b3):
    feats2 = feats.reshape(1, 256)
    w1p = jnp.zeros((128, 256), jnp.float32).at[:3].set(W1[:3])
    w1f = W1[3:]
    b1r = b1.reshape(1, 256)
    b2r = b2.reshape(1, 256)
    w3p = jnp.zeros((256, 128), jnp.float32).at[:, :1].set(W3)
    b3p = jnp.zeros((1, 128), jnp.float32).at[0, 0].set(b3[0])
    mlp = functools.partial(_mlp_eval, feats=feats2, w1p=w1p, w1f=w1f, b1=b1r,
                            w2=W2, b2=b2r, w3p=w3p, b3p=b3p)

    r0 = _RESOLUTIONS[0]
    a = jnp.linspace(0, _FINAL - 1, r0).astype(jnp.int32)
    gz, gy, gx = jnp.meshgrid(a, a, a, indexing='ij')
    coords0 = jnp.stack([gx, gy, gz], axis=0).reshape(3, -1).T
    occ0 = mlp(_pad_points(_scale(coords0)))[:r0 ** 3, 0]
    vol = occ0.reshape(r0, r0, r0)

    for res, num_pt in zip(_RESOLUTIONS[1:], _NUM_POINTS[1:]):
        stride = (_FINAL - 1) // (res - 1)
        vol = _upsample(vol)
        n = res ** 3
        flat = vol.reshape(n)
        unc = -jnp.abs(flat - 0.5)
        _, idx = jax.lax.top_k(unc, num_pt)
        xi = idx % res
        yi = (idx // res) % res
        zi = idx // (res * res)
        coords = jnp.stack([xi, yi, zi], axis=-1) * stride
        vals = mlp(_pad_points(_scale(coords)))[:num_pt, 0]
        flat = flat.at[idx].set(vals)
        vol = flat.reshape(res, res, res)

    return vol.reshape(1, 1, _FINAL, _FINAL, _FINAL)
